# Initial kernel scaffold; baseline (speedup 1.0000x reference)
import jax
import jax.numpy as jnp
from jax.experimental import pallas as pl


def kernel(users_emb, items_emb, edge_index, edge_vals):
    raise NotImplementedError("write your pallas kernel here")


# speedup vs baseline: 4.3696x; 4.3696x over previous
import functools

import jax
import jax.numpy as jnp
from jax import lax
from jax.experimental import pallas as pl
from jax.experimental.pallas import tpu as pltpu
from jax.experimental.pallas import tpu_sc as plsc

N_NODES = 10000
N_EDGES = 320000
D = 128
NC = 2
NS = 16
NW = NC * NS
E_PER_W = N_EDGES // NW
CHUNK = 80
NCHUNKS = E_PER_W // CHUNK
ROWS_PER_TILE = 624
ZROWS = 104


def _sc_partials(all_emb, src, dst, vals):
    mesh = plsc.VectorSubcoreMesh(
        core_axis_name="c", subcore_axis_name="s", num_cores=NC, num_subcores=NS
    )

    @functools.partial(
        pl.kernel,
        out_type=jax.ShapeDtypeStruct((NC * N_NODES, D), jnp.float32),
        mesh=mesh,
        scratch_types=[
            pltpu.VMEM((CHUNK,), jnp.int32),
            pltpu.VMEM((CHUNK,), jnp.int32),
            pltpu.VMEM((CHUNK,), jnp.float32),
            pltpu.VMEM((CHUNK, D), jnp.float32),
            pltpu.VMEM((ZROWS, D), jnp.float32),
            pltpu.VMEM_SHARED((N_NODES, D), jnp.float32),
            pltpu.SemaphoreType.DMA,
        ],
    )
    def body(emb_hbm, src_hbm, dst_hbm, vals_hbm, out_hbm,
             src_v, dst_v, vals_v, rows_v, zbuf, acc_sh, sem):
        cid = lax.axis_index("c")
        sid = lax.axis_index("s")
        wid = sid * NC + cid

        zeros16 = jnp.zeros((16,), jnp.float32)

        def zero_row(i, carry):
            for j in range(D // 16):
                zbuf[i, pl.ds(j * 16, 16)] = zeros16
            return carry

        lax.fori_loop(0, ZROWS, zero_row, 0)
        for kk in range(ROWS_PER_TILE // ZROWS):
            pltpu.sync_copy(
                zbuf, acc_sh.at[pl.ds(sid * ROWS_PER_TILE + kk * ZROWS, ZROWS)]
            )

        @pl.when(sid == NS - 1)
        def _zero_tail():
            pltpu.sync_copy(
                zbuf.at[pl.ds(0, 16)], acc_sh.at[pl.ds(NS * ROWS_PER_TILE, 16)]
            )

        plsc.subcore_barrier()

        def chunk_body(k, carry):
            base = wid * E_PER_W + k * CHUNK
            pltpu.sync_copy(src_hbm.at[pl.ds(base, CHUNK)], src_v)
            pltpu.sync_copy(dst_hbm.at[pl.ds(base, CHUNK)], dst_v)
            pltpu.sync_copy(vals_hbm.at[pl.ds(base, CHUNK)], vals_v)
            pltpu.async_copy(emb_hbm.at[src_v], rows_v, sem).wait()

            def group_body(g, c2):
                vv = vals_v[pl.ds(g * 16, 16)]
                for l in range(16):
                    bv = lax.gather(
                        vv,
                        jnp.full((16, 1), l, jnp.int32),
                        lax.GatherDimensionNumbers(
                            offset_dims=(),
                            collapsed_slice_dims=(0,),
                            start_index_map=(0,),
                        ),
                        slice_sizes=(1,),
                        mode=lax.GatherScatterMode.PROMISE_IN_BOUNDS,
                    )
                    r = g * 16 + l
                    for j in range(D // 16):
                        sl = pl.ds(j * 16, 16)
                        rows_v[r, sl] = rows_v[r, sl] * bv
                return c2

            lax.fori_loop(0, CHUNK // 16, group_body, 0)
            pltpu.sync_copy(rows_v, acc_sh.at[dst_v], add=True)
            return carry

        lax.fori_loop(0, NCHUNKS, chunk_body, 0)
        plsc.subcore_barrier()

        off = cid * N_NODES + sid * ROWS_PER_TILE
        pltpu.sync_copy(
            acc_sh.at[pl.ds(sid * ROWS_PER_TILE, ROWS_PER_TILE)],
            out_hbm.at[pl.ds(off, ROWS_PER_TILE)],
        )

        @pl.when(sid == NS - 1)
        def _copy_tail():
            tail = NS * ROWS_PER_TILE
            pltpu.sync_copy(
                acc_sh.at[pl.ds(tail, 16)],
                out_hbm.at[pl.ds(cid * N_NODES + tail, 16)],
            )

    return body(all_emb, src, dst, vals)


def _tc_sum(p0, p1):
    def add_body(a_ref, b_ref, o_ref):
        o_ref[...] = a_ref[...] + b_ref[...]

    blk = 1000
    return pl.pallas_call(
        add_body,
        grid=(N_NODES // blk,),
        in_specs=[
            pl.BlockSpec((blk, D), lambda i: (i, 0)),
            pl.BlockSpec((blk, D), lambda i: (i, 0)),
        ],
        out_specs=pl.BlockSpec((blk, D), lambda i: (i, 0)),
        out_shape=jax.ShapeDtypeStruct((N_NODES, D), jnp.float32),
    )(p0, p1)


def kernel(users_emb, items_emb, edge_index, edge_vals):
    num_user = users_emb.shape[0]
    all_emb = jnp.concatenate([users_emb, items_emb], axis=0)
    dst = edge_index[0].astype(jnp.int32)
    src = edge_index[1].astype(jnp.int32)
    partials = _sc_partials(all_emb, src, dst, edge_vals)
    out = _tc_sum(partials[:N_NODES], partials[N_NODES:])
    return (out[:num_user], out[num_user:])
